# Initial kernel scaffold; baseline (speedup 1.0000x reference)
import jax
import jax.numpy as jnp
from jax.experimental import pallas as pl


def kernel(x_item, x_user, edge_index_item_user, edge_index_user_item, batch_item, batch_user, proj_W_item, proj_b_item, proj_W_user, proj_b_user, bn_g_item, bn_b_item, bn_g_user, bn_b_user, convW_iu, convb_iu, rootW_iu, convW_ui, convb_ui, rootW_ui, mlp_W, mlp_b, lin_W, lin_b):
    raise NotImplementedError("write your pallas kernel here")


# speedup vs baseline: 2.8472x; 2.8472x over previous
import functools

import jax
import jax.numpy as jnp
from jax import lax
from jax.experimental import pallas as pl
from jax.experimental.pallas import tpu as pltpu
from jax.experimental.pallas import tpu_sc as plsc

N = 50000
DF = 128
H = 64
HH = 32
E = 800000
B = 32
L = 2

NC = 2
NS = 16
K = 128

CHUNKS_PER_TILE = 391
E_PAD = NS * K * CHUNKS_PER_TILE
N_ACC = 51200
ROWS_PER_TILE = N_ACC // NS
ZCHUNKS = ROWS_PER_TILE // K

_LRELU_SLOPE = 0.01
_BN_SCALE = 1.0 / (1.0 + 1e-5) ** 0.5


def _lrelu(x):
    return jnp.where(x >= 0, x, _LRELU_SLOPE * x)


def _sc_segsum_layer(h_item_lo, h_item_hi, h_user_lo, h_user_hi,
                     src_iu, dst_iu, src_ui, dst_ui):
    mesh = plsc.VectorSubcoreMesh(core_axis_name="c", subcore_axis_name="s",
                                  num_cores=NC, num_subcores=NS)

    @functools.partial(
        pl.kernel,
        out_type=[
            jax.ShapeDtypeStruct((NC, N_ACC, HH), jnp.float32),
            jax.ShapeDtypeStruct((NC, N_ACC, HH), jnp.float32),
        ],
        mesh=mesh,
        compiler_params=pltpu.CompilerParams(use_tc_tiling_on_sc=False),
        scratch_types=[
            pltpu.VMEM((K,), jnp.int32),
            pltpu.VMEM((K,), jnp.int32),
            pltpu.VMEM((K, HH), jnp.float32),
            pltpu.VMEM((K, HH), jnp.float32),
            pltpu.VMEM_SHARED((N_ACC, HH), jnp.float32),
            pltpu.SemaphoreType.DMA,
        ],
    )
    def ksc(hil, hih, hul, huh, siu, diu, sui, dui, mu_out, mi_out,
            sidx, didx, rows, zbuf, acc, sem):
        c = lax.axis_index("c")
        s = lax.axis_index("s")

        def _zb(i, carry):
            zbuf[i, pl.ds(0, 16)] = jnp.zeros((16,), jnp.float32)
            zbuf[i, pl.ds(16, 16)] = jnp.zeros((16,), jnp.float32)
            return carry
        lax.fori_loop(0, K, _zb, 0)

        def zero_acc():
            def _zc(k_, carry):
                row0 = s * ROWS_PER_TILE + k_ * K
                pltpu.sync_copy(zbuf, acc.at[pl.ds(row0, K)])
                return carry
            lax.fori_loop(0, ZCHUNKS, _zc, 0)

        def do_rel(tlo, thi, src, dst, out):
            zero_acc()
            plsc.subcore_barrier()

            def _chunk(j, carry):
                base = pl.multiple_of((s * CHUNKS_PER_TILE + j) * K, K)
                pltpu.sync_copy(src.at[pl.ds(base, K)], sidx)
                pltpu.sync_copy(dst.at[pl.ds(base, K)], didx)

                @pl.when(c == 0)
                def _():
                    pltpu.async_copy(tlo.at[sidx], rows, sem).wait()

                @pl.when(c == 1)
                def _():
                    pltpu.async_copy(thi.at[sidx], rows, sem).wait()

                pltpu.sync_copy(rows, acc.at[didx], add=True)
                return carry
            lax.fori_loop(0, CHUNKS_PER_TILE, _chunk, 0)

            plsc.subcore_barrier()
            row0 = pl.multiple_of(s * ROWS_PER_TILE, K)
            pltpu.sync_copy(acc.at[pl.ds(row0, ROWS_PER_TILE)],
                            out.at[c, pl.ds(row0, ROWS_PER_TILE)])
            plsc.subcore_barrier()

        do_rel(hil, hih, siu, diu, mu_out)
        do_rel(hul, huh, sui, dui, mi_out)

    return ksc(h_item_lo, h_item_hi, h_user_lo, h_user_hi,
               src_iu, dst_iu, src_ui, dst_ui)


_R = 2000
_G = N // _R


def _tc_proj(x_item, x_user, Wi, bi, Wu, bu, gi, bbi, gu, bbu):
    def body(xi_ref, xu_ref, wi_ref, bi_ref, wu_ref, bu_ref,
             gi_ref, bbi_ref, gu_ref, bbu_ref,
             hil_ref, hih_ref, hul_ref, huh_ref):
        hi = jnp.dot(xi_ref[...], wi_ref[...], preferred_element_type=jnp.float32)
        hi = hi + bi_ref[...][None, :]
        hi = hi * (gi_ref[...] * _BN_SCALE)[None, :] + bbi_ref[...][None, :]
        hi = _lrelu(hi)
        hil_ref[...] = hi[:, :HH]
        hih_ref[...] = hi[:, HH:]
        hu = jnp.dot(xu_ref[...], wu_ref[...], preferred_element_type=jnp.float32)
        hu = hu + bu_ref[...][None, :]
        hu = hu * (gu_ref[...] * _BN_SCALE)[None, :] + bbu_ref[...][None, :]
        hu = _lrelu(hu)
        hul_ref[...] = hu[:, :HH]
        huh_ref[...] = hu[:, HH:]

    row_spec = pl.BlockSpec((_R, DF), lambda i: (i, 0))
    w_spec = pl.BlockSpec((DF, H), lambda i: (0, 0))
    v_spec = pl.BlockSpec((H,), lambda i: (0,))
    out_spec = pl.BlockSpec((_R, HH), lambda i: (i, 0))
    return pl.pallas_call(
        body,
        grid=(_G,),
        in_specs=[row_spec, row_spec, w_spec, v_spec, w_spec, v_spec,
                  v_spec, v_spec, v_spec, v_spec],
        out_specs=[out_spec, out_spec, out_spec, out_spec],
        out_shape=[jax.ShapeDtypeStruct((N, HH), jnp.float32)] * 4,
    )(x_item, x_user, Wi, bi, Wu, bu, gi, bbi, gu, bbu)


def _tc_conv(mu, mi, hil, hih, hul, huh,
             cWiu, cbiu, rWiu, cWui, cbui, rWui, gi, bbi, gu, bbu):
    def body(mul_ref, muh_ref, mil_ref, mih_ref,
             hil_ref, hih_ref, hul_ref, huh_ref,
             cWiu_ref, cbiu_ref, rWiu_ref, cWui_ref, cbui_ref, rWui_ref,
             gi_ref, bbi_ref, gu_ref, bbu_ref,
             hil_o, hih_o, hul_o, huh_o):
        cWiu_ = cWiu_ref[...]
        rWiu_ = rWiu_ref[...]
        cWui_ = cWui_ref[...]
        rWui_ = rWui_ref[...]
        dot = functools.partial(jnp.dot, preferred_element_type=jnp.float32)
        out_u = (dot(mul_ref[0], cWiu_[:HH]) + dot(muh_ref[0], cWiu_[HH:])
                 + dot(hul_ref[...], rWiu_[:HH]) + dot(huh_ref[...], rWiu_[HH:])
                 + cbiu_ref[...][None, :])
        out_u = _lrelu(out_u * (gu_ref[...] * _BN_SCALE)[None, :] + bbu_ref[...][None, :])
        hul_o[...] = out_u[:, :HH]
        huh_o[...] = out_u[:, HH:]
        out_i = (dot(mil_ref[0], cWui_[:HH]) + dot(mih_ref[0], cWui_[HH:])
                 + dot(hil_ref[...], rWui_[:HH]) + dot(hih_ref[...], rWui_[HH:])
                 + cbui_ref[...][None, :])
        out_i = _lrelu(out_i * (gi_ref[...] * _BN_SCALE)[None, :] + bbi_ref[...][None, :])
        hil_o[...] = out_i[:, :HH]
        hih_o[...] = out_i[:, HH:]

    m_lo = pl.BlockSpec((1, _R, HH), lambda i: (0, i, 0))
    m_hi = pl.BlockSpec((1, _R, HH), lambda i: (1, i, 0))
    h_spec = pl.BlockSpec((_R, HH), lambda i: (i, 0))
    w_spec = pl.BlockSpec((H, H), lambda i: (0, 0))
    v_spec = pl.BlockSpec((H,), lambda i: (0,))
    return pl.pallas_call(
        body,
        grid=(_G,),
        in_specs=[m_lo, m_hi, m_lo, m_hi,
                  h_spec, h_spec, h_spec, h_spec,
                  w_spec, v_spec, w_spec, w_spec, v_spec, w_spec,
                  v_spec, v_spec, v_spec, v_spec],
        out_specs=[h_spec, h_spec, h_spec, h_spec],
        out_shape=[jax.ShapeDtypeStruct((N, HH), jnp.float32)] * 4,
    )(mu, mu, mi, mi, hil, hih, hul, huh,
      cWiu, cbiu, rWiu, cWui, cbui, rWui, gi, bbi, gu, bbu)


def _tc_pool_mlp(hil, hih, hul, huh, batch_item2, batch_user2,
                 mlp_W, mlp_b, lin_W, lin_b):
    NEG = -3.0e38

    def body(hil_ref, hih_ref, hul_ref, huh_ref, bi_ref, bu_ref,
             mlpW_ref, mlpb_ref, linW_ref, linb_ref,
             out_ref, sum_i, max_i, sum_u, max_u):
        step = pl.program_id(0)

        @pl.when(step == 0)
        def _():
            sum_i[...] = jnp.zeros((B, H + 8), jnp.float32)
            sum_u[...] = jnp.zeros((B, H + 8), jnp.float32)
            max_i[...] = jnp.full((B, H), NEG, jnp.float32)
            max_u[...] = jnp.full((B, H), NEG, jnp.float32)

        iota_b = lax.broadcasted_iota(jnp.int32, (1, B), 1)

        def accumulate(h_lo_ref, h_hi_ref, b_ref, sum_ref, max_ref):
            h = jnp.concatenate([h_lo_ref[...], h_hi_ref[...]], axis=1)
            bids = b_ref[...]
            oh = (bids == iota_b).astype(jnp.float32)
            hx = jnp.concatenate(
                [h, jnp.ones((_R, 8), jnp.float32)], axis=1)
            sum_ref[...] += lax.dot_general(
                oh, hx, (((0,), (0,)), ((), ())),
                preferred_element_type=jnp.float32)
            for b in range(B):
                mrow = jnp.max(jnp.where(bids == b, h, NEG), axis=0)
                max_ref[b, :] = jnp.maximum(max_ref[b, :], mrow)

        accumulate(hil_ref, hih_ref, bi_ref, sum_i, max_i)
        accumulate(hul_ref, huh_ref, bu_ref, sum_u, max_u)

        @pl.when(step == _G - 1)
        def _():
            def finish(sum_ref, max_ref):
                cnt = sum_ref[:, H:H + 1]
                mean = sum_ref[:, :H] / jnp.maximum(cnt, 1.0)
                mean = jnp.where(cnt > 0, mean, 0.0)
                mx = jnp.where(cnt > 0, max_ref[...], 0.0)
                return mean, mx
            mean_i, mx_i = finish(sum_i, max_i)
            mean_u, mx_u = finish(sum_u, max_u)
            rep = jnp.concatenate([mean_i, mx_i, mean_u, mx_u], axis=1)
            hid = jnp.dot(rep, mlpW_ref[...], preferred_element_type=jnp.float32)
            hid = hid + mlpb_ref[...][None, :]
            out = jnp.sum(hid * linW_ref[...][:, 0][None, :], axis=1,
                          keepdims=True) + linb_ref[...][None, :]
            out_ref[...] = out

    h_spec = pl.BlockSpec((_R, HH), lambda i: (i, 0))
    b_spec = pl.BlockSpec((_R, 1), lambda i: (i, 0))
    return pl.pallas_call(
        body,
        grid=(_G,),
        in_specs=[h_spec, h_spec, h_spec, h_spec, b_spec, b_spec,
                  pl.BlockSpec((4 * H, H), lambda i: (0, 0)),
                  pl.BlockSpec((H,), lambda i: (0,)),
                  pl.BlockSpec((H, 1), lambda i: (0, 0)),
                  pl.BlockSpec((1,), lambda i: (0,))],
        out_specs=pl.BlockSpec((B, 1), lambda i: (0, 0)),
        out_shape=jax.ShapeDtypeStruct((B, 1), jnp.float32),
        scratch_shapes=[pltpu.VMEM((B, H + 8), jnp.float32),
                        pltpu.VMEM((B, H), jnp.float32),
                        pltpu.VMEM((B, H + 8), jnp.float32),
                        pltpu.VMEM((B, H), jnp.float32)],
    )(hil, hih, hul, huh, batch_item2, batch_user2,
      mlp_W, mlp_b, lin_W, lin_b)


def kernel(x_item, x_user, edge_index_item_user, edge_index_user_item,
           batch_item, batch_user,
           proj_W_item, proj_b_item, proj_W_user, proj_b_user,
           bn_g_item, bn_b_item, bn_g_user, bn_b_user,
           convW_iu, convb_iu, rootW_iu,
           convW_ui, convb_ui, rootW_ui,
           mlp_W, mlp_b, lin_W, lin_b):
    pad = E_PAD - E
    pad_src = jnp.zeros((pad,), jnp.int32)
    pad_dst = jnp.full((pad,), N, jnp.int32)
    src_iu = jnp.concatenate([edge_index_item_user[0], pad_src])
    dst_iu = jnp.concatenate([edge_index_item_user[1], pad_dst])
    src_ui = jnp.concatenate([edge_index_user_item[0], pad_src])
    dst_ui = jnp.concatenate([edge_index_user_item[1], pad_dst])

    hil, hih, hul, huh = _tc_proj(
        x_item, x_user, proj_W_item, proj_b_item, proj_W_user, proj_b_user,
        bn_g_item, bn_b_item, bn_g_user, bn_b_user)

    for l in range(L):
        mu, mi = _sc_segsum_layer(hil, hih, hul, huh,
                                  src_iu, dst_iu, src_ui, dst_ui)
        hil, hih, hul, huh = _tc_conv(
            mu, mi, hil, hih, hul, huh,
            convW_iu[l], convb_iu[l], rootW_iu[l],
            convW_ui[l], convb_ui[l], rootW_ui[l],
            bn_g_item, bn_b_item, bn_g_user, bn_b_user)

    batch_item2 = batch_item.reshape(N, 1)
    batch_user2 = batch_user.reshape(N, 1)
    return _tc_pool_mlp(hil, hih, hul, huh, batch_item2, batch_user2,
                        mlp_W, mlp_b, lin_W, lin_b)


# speedup vs baseline: 6.3858x; 2.2428x over previous
import functools

import jax
import jax.numpy as jnp
from jax import lax
from jax.experimental import pallas as pl
from jax.experimental.pallas import tpu as pltpu
from jax.experimental.pallas import tpu_sc as plsc

N = 50000
DF = 128
H = 64
HH = 32
E = 800000
B = 32
L = 2

NC = 2
NS = 16
K = 128
NBUF = 4

NCH = 392
IB = 28
NG = NCH // IB
E_PAD = NS * K * NCH
N_ACC = 51200
ROWS_PER_TILE = N_ACC // NS
ZCHUNKS = ROWS_PER_TILE // K

_LRELU_SLOPE = 0.01
_BN_SCALE = 1.0 / (1.0 + 1e-5) ** 0.5


def _lrelu(x):
    return jnp.where(x >= 0, x, _LRELU_SLOPE * x)


def _sc_segsum_layer(h_item, h_user, eidx_iu, eidx_ui):
    mesh = plsc.VectorSubcoreMesh(core_axis_name="c", subcore_axis_name="s",
                                  num_cores=NC, num_subcores=NS)

    @functools.partial(
        pl.kernel,
        out_type=[
            jax.ShapeDtypeStruct((NC, N_ACC, HH), jnp.float32),
            jax.ShapeDtypeStruct((NC, N_ACC, HH), jnp.float32),
        ],
        mesh=mesh,
        compiler_params=pltpu.CompilerParams(use_tc_tiling_on_sc=False),
        scratch_types=[
            pltpu.VMEM((IB, K), jnp.int32),
            pltpu.VMEM((IB, K), jnp.int32),
            pltpu.VMEM((NBUF, K, HH), jnp.float32),
            pltpu.VMEM((K, HH), jnp.float32),
            pltpu.VMEM_SHARED((N_ACC, HH), jnp.float32),
            pltpu.SemaphoreType.DMA((NBUF,)),
        ],
    )
    def ksc(h_item_r, h_user_r, eiu_r, eui_r, mu_out, mi_out,
            sidx, didx, rows, zbuf, acc, gsems):
        c = lax.axis_index("c")
        s = lax.axis_index("s")

        def _zb(i, carry):
            zbuf[i, pl.ds(0, 16)] = jnp.zeros((16,), jnp.float32)
            zbuf[i, pl.ds(16, 16)] = jnp.zeros((16,), jnp.float32)
            return carry
        lax.fori_loop(0, K, _zb, 0)

        def zero_acc():
            def _zc(k_, carry):
                row0 = s * ROWS_PER_TILE + k_ * K
                pltpu.sync_copy(zbuf, acc.at[pl.ds(row0, K)])
                return carry
            lax.fori_loop(0, ZCHUNKS, _zc, 0)

        def do_rel(tbl, edges, out):
            zero_acc()
            plsc.subcore_barrier()

            def start_gather(jj, b):
                pltpu.async_copy(tbl.at[c].at[sidx.at[jj]], rows.at[b],
                                 gsems.at[b])

            def wait_gather(b):
                pltpu.make_async_copy(tbl.at[c].at[sidx.at[0]], rows.at[b],
                                      gsems.at[b]).wait()

            def group(g, carry):
                pltpu.sync_copy(edges.at[0, s, pl.ds(g * IB, IB)], sidx)
                pltpu.sync_copy(edges.at[1, s, pl.ds(g * IB, IB)], didx)
                for b in range(NBUF):
                    start_gather(b, b)

                def ring(r, carry2):
                    for b in range(NBUF):
                        jj = r * NBUF + b
                        wait_gather(b)
                        pltpu.sync_copy(rows.at[b], acc.at[didx.at[jj]],
                                        add=True)

                        @pl.when(jj + NBUF < IB)
                        def _():
                            start_gather(jj + NBUF, b)
                    return carry2
                lax.fori_loop(0, IB // NBUF, ring, 0)
                return carry
            lax.fori_loop(0, NG, group, 0)

            plsc.subcore_barrier()
            row0 = s * ROWS_PER_TILE
            pltpu.sync_copy(acc.at[pl.ds(row0, ROWS_PER_TILE)],
                            out.at[c, pl.ds(row0, ROWS_PER_TILE)])
            plsc.subcore_barrier()

        do_rel(h_item_r, eiu_r, mu_out)
        do_rel(h_user_r, eui_r, mi_out)

    return ksc(h_item, h_user, eidx_iu, eidx_ui)


_R = 2000
_G = N // _R


def _split_store(h, out_ref):
    out_ref[0] = h[:, :HH]
    out_ref[1] = h[:, HH:]


def _tc_proj(x_item, x_user, Wi, bi, Wu, bu, gi, bbi, gu, bbu):
    def body(xi_ref, xu_ref, wi_ref, bi_ref, wu_ref, bu_ref,
             gi_ref, bbi_ref, gu_ref, bbu_ref, hi_ref, hu_ref):
        hi = jnp.dot(xi_ref[...], wi_ref[...], preferred_element_type=jnp.float32)
        hi = hi + bi_ref[...][None, :]
        hi = hi * (gi_ref[...] * _BN_SCALE)[None, :] + bbi_ref[...][None, :]
        _split_store(_lrelu(hi), hi_ref)
        hu = jnp.dot(xu_ref[...], wu_ref[...], preferred_element_type=jnp.float32)
        hu = hu + bu_ref[...][None, :]
        hu = hu * (gu_ref[...] * _BN_SCALE)[None, :] + bbu_ref[...][None, :]
        _split_store(_lrelu(hu), hu_ref)

    row_spec = pl.BlockSpec((_R, DF), lambda i: (i, 0))
    w_spec = pl.BlockSpec((DF, H), lambda i: (0, 0))
    v_spec = pl.BlockSpec((H,), lambda i: (0,))
    out_spec = pl.BlockSpec((NC, _R, HH), lambda i: (0, i, 0))
    return pl.pallas_call(
        body,
        grid=(_G,),
        in_specs=[row_spec, row_spec, w_spec, v_spec, w_spec, v_spec,
                  v_spec, v_spec, v_spec, v_spec],
        out_specs=[out_spec, out_spec],
        out_shape=[jax.ShapeDtypeStruct((NC, N, HH), jnp.float32)] * 2,
    )(x_item, x_user, Wi, bi, Wu, bu, gi, bbi, gu, bbu)


def _tc_conv(mu, mi, h_item, h_user,
             cWiu, cbiu, rWiu, cWui, cbui, rWui, gi, bbi, gu, bbu):
    def body(mu_ref, mi_ref, hi_ref, hu_ref,
             cWiu_ref, cbiu_ref, rWiu_ref, cWui_ref, cbui_ref, rWui_ref,
             gi_ref, bbi_ref, gu_ref, bbu_ref, hi_o, hu_o):
        dot = functools.partial(jnp.dot, preferred_element_type=jnp.float32)
        cWiu_ = cWiu_ref[...]
        rWiu_ = rWiu_ref[...]
        cWui_ = cWui_ref[...]
        rWui_ = rWui_ref[...]
        out_u = (dot(mu_ref[0], cWiu_[:HH]) + dot(mu_ref[1], cWiu_[HH:])
                 + dot(hu_ref[0], rWiu_[:HH]) + dot(hu_ref[1], rWiu_[HH:])
                 + cbiu_ref[...][None, :])
        out_u = _lrelu(out_u * (gu_ref[...] * _BN_SCALE)[None, :]
                       + bbu_ref[...][None, :])
        _split_store(out_u, hu_o)
        out_i = (dot(mi_ref[0], cWui_[:HH]) + dot(mi_ref[1], cWui_[HH:])
                 + dot(hi_ref[0], rWui_[:HH]) + dot(hi_ref[1], rWui_[HH:])
                 + cbui_ref[...][None, :])
        out_i = _lrelu(out_i * (gi_ref[...] * _BN_SCALE)[None, :]
                       + bbi_ref[...][None, :])
        _split_store(out_i, hi_o)

    m_spec = pl.BlockSpec((NC, _R, HH), lambda i: (0, i, 0))
    h_spec = pl.BlockSpec((NC, _R, HH), lambda i: (0, i, 0))
    w_spec = pl.BlockSpec((H, H), lambda i: (0, 0))
    v_spec = pl.BlockSpec((H,), lambda i: (0,))
    return pl.pallas_call(
        body,
        grid=(_G,),
        in_specs=[m_spec, m_spec, h_spec, h_spec,
                  w_spec, v_spec, w_spec, w_spec, v_spec, w_spec,
                  v_spec, v_spec, v_spec, v_spec],
        out_specs=[h_spec, h_spec],
        out_shape=[jax.ShapeDtypeStruct((NC, N, HH), jnp.float32)] * 2,
    )(mu, mi, h_item, h_user,
      cWiu, cbiu, rWiu, cWui, cbui, rWui, gi, bbi, gu, bbu)


def _tc_pool_mlp(h_item, h_user, batch_item2, batch_user2,
                 mlp_W, mlp_b, lin_W, lin_b):
    NEG = -3.0e38

    def body(hi_ref, hu_ref, bi_ref, bu_ref,
             mlpW_ref, mlpb_ref, linW_ref, linb_ref,
             out_ref, sum_i, max_i, sum_u, max_u):
        step = pl.program_id(0)

        @pl.when(step == 0)
        def _():
            sum_i[...] = jnp.zeros((B, H + 8), jnp.float32)
            sum_u[...] = jnp.zeros((B, H + 8), jnp.float32)
            max_i[...] = jnp.full((B, H), NEG, jnp.float32)
            max_u[...] = jnp.full((B, H), NEG, jnp.float32)

        iota_b = lax.broadcasted_iota(jnp.int32, (1, B), 1)

        def accumulate(h_ref, b_ref, sum_ref, max_ref):
            h = jnp.concatenate([h_ref[0], h_ref[1]], axis=1)
            bids = b_ref[...]
            oh = (bids == iota_b).astype(jnp.float32)
            hx = jnp.concatenate(
                [h, jnp.ones((_R, 8), jnp.float32)], axis=1)
            sum_ref[...] += lax.dot_general(
                oh, hx, (((0,), (0,)), ((), ())),
                preferred_element_type=jnp.float32)
            for b in range(B):
                mrow = jnp.max(jnp.where(bids == b, h, NEG), axis=0)
                max_ref[b, :] = jnp.maximum(max_ref[b, :], mrow)

        accumulate(hi_ref, bi_ref, sum_i, max_i)
        accumulate(hu_ref, bu_ref, sum_u, max_u)

        @pl.when(step == _G - 1)
        def _():
            def finish(sum_ref, max_ref):
                cnt = sum_ref[:, H:H + 1]
                mean = sum_ref[:, :H] / jnp.maximum(cnt, 1.0)
                mean = jnp.where(cnt > 0, mean, 0.0)
                mx = jnp.where(cnt > 0, max_ref[...], 0.0)
                return mean, mx
            mean_i, mx_i = finish(sum_i, max_i)
            mean_u, mx_u = finish(sum_u, max_u)
            rep = jnp.concatenate([mean_i, mx_i, mean_u, mx_u], axis=1)
            hid = jnp.dot(rep, mlpW_ref[...], preferred_element_type=jnp.float32)
            hid = hid + mlpb_ref[...][None, :]
            out = jnp.sum(hid * linW_ref[...][:, 0][None, :], axis=1,
                          keepdims=True) + linb_ref[...][None, :]
            out_ref[...] = out

    h_spec = pl.BlockSpec((NC, _R, HH), lambda i: (0, i, 0))
    b_spec = pl.BlockSpec((_R, 1), lambda i: (i, 0))
    return pl.pallas_call(
        body,
        grid=(_G,),
        in_specs=[h_spec, h_spec, b_spec, b_spec,
                  pl.BlockSpec((4 * H, H), lambda i: (0, 0)),
                  pl.BlockSpec((H,), lambda i: (0,)),
                  pl.BlockSpec((H, 1), lambda i: (0, 0)),
                  pl.BlockSpec((1,), lambda i: (0,))],
        out_specs=pl.BlockSpec((B, 1), lambda i: (0, 0)),
        out_shape=jax.ShapeDtypeStruct((B, 1), jnp.float32),
        scratch_shapes=[pltpu.VMEM((B, H + 8), jnp.float32),
                        pltpu.VMEM((B, H), jnp.float32),
                        pltpu.VMEM((B, H + 8), jnp.float32),
                        pltpu.VMEM((B, H), jnp.float32)],
    )(h_item, h_user, batch_item2, batch_user2,
      mlp_W, mlp_b, lin_W, lin_b)


def _pad_edges(edge_index):
    pad = E_PAD - E
    src = jnp.concatenate([edge_index[0], jnp.zeros((pad,), jnp.int32)])
    dst = jnp.concatenate([edge_index[1], jnp.full((pad,), N, jnp.int32)])
    return jnp.stack([src.reshape(NS, NCH, K), dst.reshape(NS, NCH, K)])


def kernel(x_item, x_user, edge_index_item_user, edge_index_user_item,
           batch_item, batch_user,
           proj_W_item, proj_b_item, proj_W_user, proj_b_user,
           bn_g_item, bn_b_item, bn_g_user, bn_b_user,
           convW_iu, convb_iu, rootW_iu,
           convW_ui, convb_ui, rootW_ui,
           mlp_W, mlp_b, lin_W, lin_b):
    eidx_iu = _pad_edges(edge_index_item_user)
    eidx_ui = _pad_edges(edge_index_user_item)

    h_item, h_user = _tc_proj(
        x_item, x_user, proj_W_item, proj_b_item, proj_W_user, proj_b_user,
        bn_g_item, bn_b_item, bn_g_user, bn_b_user)

    for l in range(L):
        mu, mi = _sc_segsum_layer(h_item, h_user, eidx_iu, eidx_ui)
        h_item, h_user = _tc_conv(
            mu, mi, h_item, h_user,
            convW_iu[l], convb_iu[l], rootW_iu[l],
            convW_ui[l], convb_ui[l], rootW_ui[l],
            bn_g_item, bn_b_item, bn_g_user, bn_b_user)

    batch_item2 = batch_item.reshape(N, 1)
    batch_user2 = batch_user.reshape(N, 1)
    return _tc_pool_mlp(h_item, h_user, batch_item2, batch_user2,
                        mlp_W, mlp_b, lin_W, lin_b)


# speedup vs baseline: 7.9817x; 1.2499x over previous
import functools

import jax
import jax.numpy as jnp
from jax import lax
from jax.experimental import pallas as pl
from jax.experimental.pallas import tpu as pltpu
from jax.experimental.pallas import tpu_sc as plsc

N = 50000
DF = 128
H = 64
HH = 32
E = 800000
B = 32
L = 2

NC = 2
NS = 16
K = 128
NBUF = 4

NCH = 392
IB = 28
NG = NCH // IB
E_PAD = NS * K * NCH
N_ACC = 51200
ROWS_PER_TILE = N_ACC // NS
ZCHUNKS = ROWS_PER_TILE // K

_LRELU_SLOPE = 0.01
_BN_SCALE = 1.0 / (1.0 + 1e-5) ** 0.5


def _lrelu(x):
    return jnp.where(x >= 0, x, _LRELU_SLOPE * x)


def _sc_segsum_layer(h_item, h_user, eidx_iu, eidx_ui):
    mesh = plsc.VectorSubcoreMesh(core_axis_name="c", subcore_axis_name="s",
                                  num_cores=NC, num_subcores=NS)

    @functools.partial(
        pl.kernel,
        out_type=[
            jax.ShapeDtypeStruct((NC, N_ACC, HH), jnp.float32),
            jax.ShapeDtypeStruct((NC, N_ACC, HH), jnp.float32),
        ],
        mesh=mesh,
        compiler_params=pltpu.CompilerParams(use_tc_tiling_on_sc=False),
        scratch_types=[
            pltpu.VMEM((IB, K), jnp.int32),
            pltpu.VMEM((IB, K), jnp.int32),
            pltpu.VMEM((NBUF, K, HH), jnp.float32),
            pltpu.VMEM((K, HH), jnp.float32),
            pltpu.VMEM_SHARED((N_ACC, HH), jnp.float32),
            pltpu.SemaphoreType.DMA((NBUF,)),
        ],
    )
    def ksc(h_item_r, h_user_r, eiu_r, eui_r, mu_out, mi_out,
            sidx, didx, rows, zbuf, acc, gsems):
        c = lax.axis_index("c")
        s = lax.axis_index("s")

        def _zb(i, carry):
            zbuf[i, pl.ds(0, 16)] = jnp.zeros((16,), jnp.float32)
            zbuf[i, pl.ds(16, 16)] = jnp.zeros((16,), jnp.float32)
            return carry
        lax.fori_loop(0, K, _zb, 0)

        def zero_acc():
            def _zc(k_, carry):
                row0 = s * ROWS_PER_TILE + k_ * K
                pltpu.sync_copy(zbuf, acc.at[pl.ds(row0, K)])
                return carry
            lax.fori_loop(0, ZCHUNKS, _zc, 0)

        def do_rel(tbl, edges, out):
            zero_acc()
            plsc.subcore_barrier()

            def start_gather(jj, b):
                pltpu.async_copy(tbl.at[c].at[sidx.at[jj]], rows.at[b],
                                 gsems.at[b])

            def wait_gather(b):
                pltpu.make_async_copy(tbl.at[c].at[sidx.at[0]], rows.at[b],
                                      gsems.at[b]).wait()

            def group(g, carry):
                pltpu.sync_copy(edges.at[0, s, pl.ds(g * IB, IB)], sidx)
                pltpu.sync_copy(edges.at[1, s, pl.ds(g * IB, IB)], didx)
                for b in range(NBUF):
                    start_gather(b, b)

                def ring(r, carry2):
                    for b in range(NBUF):
                        jj = r * NBUF + b
                        wait_gather(b)
                        pltpu.sync_copy(rows.at[b], acc.at[didx.at[jj]],
                                        add=True)

                        @pl.when(jj + NBUF < IB)
                        def _():
                            start_gather(jj + NBUF, b)
                    return carry2
                lax.fori_loop(0, IB // NBUF, ring, 0)
                return carry
            lax.fori_loop(0, NG, group, 0)

            plsc.subcore_barrier()
            row0 = s * ROWS_PER_TILE
            pltpu.sync_copy(acc.at[pl.ds(row0, ROWS_PER_TILE)],
                            out.at[c, pl.ds(row0, ROWS_PER_TILE)])
            plsc.subcore_barrier()

        do_rel(h_item_r, eiu_r, mu_out)
        do_rel(h_user_r, eui_r, mi_out)

    return ksc(h_item, h_user, eidx_iu, eidx_ui)


_R = 2000
_G = N // _R


def _split_store(h, out_ref):
    out_ref[0] = h[:, :HH]
    out_ref[1] = h[:, HH:]


def _tc_proj(x_item, x_user, Wi, bi, Wu, bu, gi, bbi, gu, bbu):
    def body(xi_ref, xu_ref, wi_ref, bi_ref, wu_ref, bu_ref,
             gi_ref, bbi_ref, gu_ref, bbu_ref, hi_ref, hu_ref):
        hi = jnp.dot(xi_ref[...], wi_ref[...], preferred_element_type=jnp.float32)
        hi = hi + bi_ref[...][None, :]
        hi = hi * (gi_ref[...] * _BN_SCALE)[None, :] + bbi_ref[...][None, :]
        _split_store(_lrelu(hi), hi_ref)
        hu = jnp.dot(xu_ref[...], wu_ref[...], preferred_element_type=jnp.float32)
        hu = hu + bu_ref[...][None, :]
        hu = hu * (gu_ref[...] * _BN_SCALE)[None, :] + bbu_ref[...][None, :]
        _split_store(_lrelu(hu), hu_ref)

    row_spec = pl.BlockSpec((_R, DF), lambda i: (i, 0))
    w_spec = pl.BlockSpec((DF, H), lambda i: (0, 0))
    v_spec = pl.BlockSpec((H,), lambda i: (0,))
    out_spec = pl.BlockSpec((NC, _R, HH), lambda i: (0, i, 0))
    return pl.pallas_call(
        body,
        grid=(_G,),
        in_specs=[row_spec, row_spec, w_spec, v_spec, w_spec, v_spec,
                  v_spec, v_spec, v_spec, v_spec],
        out_specs=[out_spec, out_spec],
        out_shape=[jax.ShapeDtypeStruct((NC, N, HH), jnp.float32)] * 2,
    )(x_item, x_user, Wi, bi, Wu, bu, gi, bbi, gu, bbu)


def _tc_conv(mu, mi, h_item, h_user,
             cWiu, cbiu, rWiu, cWui, cbui, rWui, gi, bbi, gu, bbu):
    def body(mu_ref, mi_ref, hi_ref, hu_ref,
             cWiu_ref, cbiu_ref, rWiu_ref, cWui_ref, cbui_ref, rWui_ref,
             gi_ref, bbi_ref, gu_ref, bbu_ref, hi_o, hu_o):
        dot = functools.partial(jnp.dot, preferred_element_type=jnp.float32)
        cWiu_ = cWiu_ref[...]
        rWiu_ = rWiu_ref[...]
        cWui_ = cWui_ref[...]
        rWui_ = rWui_ref[...]
        out_u = (dot(mu_ref[0], cWiu_[:HH]) + dot(mu_ref[1], cWiu_[HH:])
                 + dot(hu_ref[0], rWiu_[:HH]) + dot(hu_ref[1], rWiu_[HH:])
                 + cbiu_ref[...][None, :])
        out_u = _lrelu(out_u * (gu_ref[...] * _BN_SCALE)[None, :]
                       + bbu_ref[...][None, :])
        _split_store(out_u, hu_o)
        out_i = (dot(mi_ref[0], cWui_[:HH]) + dot(mi_ref[1], cWui_[HH:])
                 + dot(hi_ref[0], rWui_[:HH]) + dot(hi_ref[1], rWui_[HH:])
                 + cbui_ref[...][None, :])
        out_i = _lrelu(out_i * (gi_ref[...] * _BN_SCALE)[None, :]
                       + bbi_ref[...][None, :])
        _split_store(out_i, hi_o)

    m_spec = pl.BlockSpec((NC, _R, HH), lambda i: (0, i, 0))
    h_spec = pl.BlockSpec((NC, _R, HH), lambda i: (0, i, 0))
    w_spec = pl.BlockSpec((H, H), lambda i: (0, 0))
    v_spec = pl.BlockSpec((H,), lambda i: (0,))
    return pl.pallas_call(
        body,
        grid=(_G,),
        in_specs=[m_spec, m_spec, h_spec, h_spec,
                  w_spec, v_spec, w_spec, w_spec, v_spec, w_spec,
                  v_spec, v_spec, v_spec, v_spec],
        out_specs=[h_spec, h_spec],
        out_shape=[jax.ShapeDtypeStruct((NC, N, HH), jnp.float32)] * 2,
    )(mu, mi, h_item, h_user,
      cWiu, cbiu, rWiu, cWui, cbui, rWui, gi, bbi, gu, bbu)


def _tc_pool_mlp(h_item, h_user, batch_item2, batch_user2,
                 mlp_W, mlp_b, lin_W, lin_b):
    NEG = -3.0e38

    def body(hi_ref, hu_ref, bi_ref, bu_ref,
             mlpW_ref, mlpb_ref, linW_ref, linb_ref,
             out_ref, sum_i, max_i, sum_u, max_u):
        step = pl.program_id(0)

        @pl.when(step == 0)
        def _():
            sum_i[...] = jnp.zeros((B, H + 8), jnp.float32)
            sum_u[...] = jnp.zeros((B, H + 8), jnp.float32)
            max_i[...] = jnp.full((B, H), NEG, jnp.float32)
            max_u[...] = jnp.full((B, H), NEG, jnp.float32)

        iota_b = lax.broadcasted_iota(jnp.int32, (1, B), 1)

        def accumulate(h_ref, b_ref, sum_ref, max_ref):
            h = jnp.concatenate([h_ref[0], h_ref[1]], axis=1)
            bids = b_ref[...]
            oh = (bids == iota_b).astype(jnp.float32)
            hx = jnp.concatenate(
                [h, jnp.ones((_R, 8), jnp.float32)], axis=1)
            sum_ref[...] += lax.dot_general(
                oh, hx, (((0,), (0,)), ((), ())),
                preferred_element_type=jnp.float32)
            bmin = b_ref[0, 0]
            bmax = b_ref[_R - 1, 0]
            for b in range(B):
                @pl.when((bmin <= b) & (b <= bmax))
                def _():
                    mrow = jnp.max(jnp.where(bids == b, h, NEG), axis=0)
                    max_ref[b, :] = jnp.maximum(max_ref[b, :], mrow)

        accumulate(hi_ref, bi_ref, sum_i, max_i)
        accumulate(hu_ref, bu_ref, sum_u, max_u)

        @pl.when(step == _G - 1)
        def _():
            def finish(sum_ref, max_ref):
                cnt = sum_ref[:, H:H + 1]
                mean = sum_ref[:, :H] / jnp.maximum(cnt, 1.0)
                mean = jnp.where(cnt > 0, mean, 0.0)
                mx = jnp.where(cnt > 0, max_ref[...], 0.0)
                return mean, mx
            mean_i, mx_i = finish(sum_i, max_i)
            mean_u, mx_u = finish(sum_u, max_u)
            rep = jnp.concatenate([mean_i, mx_i, mean_u, mx_u], axis=1)
            hid = jnp.dot(rep, mlpW_ref[...], preferred_element_type=jnp.float32)
            hid = hid + mlpb_ref[...][None, :]
            out = jnp.sum(hid * linW_ref[...][:, 0][None, :], axis=1,
                          keepdims=True) + linb_ref[...][None, :]
            out_ref[...] = out

    h_spec = pl.BlockSpec((NC, _R, HH), lambda i: (0, i, 0))
    b_spec = pl.BlockSpec((_R, 1), lambda i: (i, 0))
    return pl.pallas_call(
        body,
        grid=(_G,),
        in_specs=[h_spec, h_spec, b_spec, b_spec,
                  pl.BlockSpec((4 * H, H), lambda i: (0, 0)),
                  pl.BlockSpec((H,), lambda i: (0,)),
                  pl.BlockSpec((H, 1), lambda i: (0, 0)),
                  pl.BlockSpec((1,), lambda i: (0,))],
        out_specs=pl.BlockSpec((B, 1), lambda i: (0, 0)),
        out_shape=jax.ShapeDtypeStruct((B, 1), jnp.float32),
        scratch_shapes=[pltpu.VMEM((B, H + 8), jnp.float32),
                        pltpu.VMEM((B, H), jnp.float32),
                        pltpu.VMEM((B, H + 8), jnp.float32),
                        pltpu.VMEM((B, H), jnp.float32)],
    )(h_item, h_user, batch_item2, batch_user2,
      mlp_W, mlp_b, lin_W, lin_b)


def _pad_edges(edge_index):
    pad = E_PAD - E
    src = jnp.concatenate([edge_index[0], jnp.zeros((pad,), jnp.int32)])
    dst = jnp.concatenate([edge_index[1], jnp.full((pad,), N, jnp.int32)])
    return jnp.stack([src.reshape(NS, NCH, K), dst.reshape(NS, NCH, K)])


def kernel(x_item, x_user, edge_index_item_user, edge_index_user_item,
           batch_item, batch_user,
           proj_W_item, proj_b_item, proj_W_user, proj_b_user,
           bn_g_item, bn_b_item, bn_g_user, bn_b_user,
           convW_iu, convb_iu, rootW_iu,
           convW_ui, convb_ui, rootW_ui,
           mlp_W, mlp_b, lin_W, lin_b):
    eidx_iu = _pad_edges(edge_index_item_user)
    eidx_ui = _pad_edges(edge_index_user_item)

    h_item, h_user = _tc_proj(
        x_item, x_user, proj_W_item, proj_b_item, proj_W_user, proj_b_user,
        bn_g_item, bn_b_item, bn_g_user, bn_b_user)

    for l in range(L):
        mu, mi = _sc_segsum_layer(h_item, h_user, eidx_iu, eidx_ui)
        h_item, h_user = _tc_conv(
            mu, mi, h_item, h_user,
            convW_iu[l], convb_iu[l], rootW_iu[l],
            convW_ui[l], convb_ui[l], rootW_ui[l],
            bn_g_item, bn_b_item, bn_g_user, bn_b_user)

    batch_item2 = batch_item.reshape(N, 1)
    batch_user2 = batch_user.reshape(N, 1)
    return _tc_pool_mlp(h_item, h_user, batch_item2, batch_user2,
                        mlp_W, mlp_b, lin_W, lin_b)
